# baseline (speedup 1.0000x reference)
import functools
import math

import jax
import jax.numpy as jnp
from jax import lax
from jax.experimental import pallas as pl
from jax.experimental.pallas import tpu as pltpu
from jax.experimental.pallas import tpu_sc as plsc

B = 1
S = 2048
D = 1024
H = 16
KVH = 8
DK = D // H
HID = 1024
E = 8
TOPK = 2
NSH = 1
THETA = 10000.0
EPS = 1e-6

BS = 256
QB = 512
LANES = 128

T = S
NP = T * TOPK
BM = 256
NPAD = 5888
NBLK = NPAD // BM
NW = 32


def _rms(x, g):
    return g * (x / jnp.sqrt(jnp.mean(x * x, axis=-1, keepdims=True) + EPS))


def _k1_body(x_ref, g_ref, wq_ref, wkv_ref, cos_ref, sin_ref, q_ref, k_ref, v_ref):
    xn = _rms(x_ref[...], g_ref[...]).astype(jnp.bfloat16)
    q = jnp.dot(xn, wq_ref[...], preferred_element_type=jnp.float32)
    kv = jnp.dot(xn, wkv_ref[...], preferred_element_type=jnp.float32)
    k = kv[:, : KVH * DK]
    v = kv[:, KVH * DK :]
    cos = cos_ref[...]
    sin = sin_ref[...]

    def rope(t, c, s):
        even = jax.lax.broadcasted_iota(jnp.int32, t.shape, 1) % 2 == 0
        n = t.shape[1]
        swap = jnp.where(even, pltpu.roll(t, n - 1, 1), pltpu.roll(t, 1, 1))
        return t * c + swap * s

    qr = rope(q, cos, sin) * (1.0 / math.sqrt(DK))
    kr = rope(k, cos[:, : KVH * DK], sin[:, : KVH * DK])
    for hh in range(H):
        q_ref[hh] = qr[:, hh * DK : (hh + 1) * DK]
    for hh in range(KVH):
        k_ref[hh] = kr[:, hh * DK : (hh + 1) * DK]
        v_ref[hh] = v[:, hh * DK : (hh + 1) * DK]


def _k1(x2, g_attn, Wq, Wkv, cosD, sinD):
    return pl.pallas_call(
        _k1_body,
        grid=(S // BS,),
        in_specs=[
            pl.BlockSpec((BS, D), lambda i: (i, 0)),
            pl.BlockSpec((D,), lambda i: (0,)),
            pl.BlockSpec((D, D), lambda i: (0, 0)),
            pl.BlockSpec((D, D), lambda i: (0, 0)),
            pl.BlockSpec((BS, D), lambda i: (i, 0)),
            pl.BlockSpec((BS, D), lambda i: (i, 0)),
        ],
        out_specs=[
            pl.BlockSpec((H, BS, DK), lambda i: (0, i, 0)),
            pl.BlockSpec((KVH, BS, DK), lambda i: (0, i, 0)),
            pl.BlockSpec((KVH, BS, DK), lambda i: (0, i, 0)),
        ],
        out_shape=[
            jax.ShapeDtypeStruct((H, S, DK), jnp.float32),
            jax.ShapeDtypeStruct((KVH, S, DK), jnp.float32),
            jax.ShapeDtypeStruct((KVH, S, DK), jnp.float32),
        ],
        compiler_params=pltpu.CompilerParams(
            dimension_semantics=("arbitrary",)),
    )(x2, g_attn, Wq, Wkv, cosD, sinD)


def _k2_body(q_ref, k_ref, v_ref, o_ref):
    sb = pl.program_id(1)
    q = q_ref[0]
    k = k_ref[0]
    v = v_ref[0]
    logits = jax.lax.dot_general(q, k, (((1,), (1,)), ((), ())),
                                 preferred_element_type=jnp.float32)
    row = jax.lax.broadcasted_iota(jnp.int32, logits.shape, 0) + sb * QB
    col = jax.lax.broadcasted_iota(jnp.int32, logits.shape, 1)
    logits = jnp.where(col <= row, logits, -1e30)
    m = jnp.max(logits, axis=-1, keepdims=True)
    p = jnp.exp(logits - m)
    p = p / jnp.sum(p, axis=-1, keepdims=True)
    o_ref[0] = jnp.dot(p, v, preferred_element_type=jnp.float32)


def _k2(q, k, v):
    return pl.pallas_call(
        _k2_body,
        grid=(H, S // QB),
        in_specs=[
            pl.BlockSpec((1, QB, DK), lambda h, sb: (h, sb, 0)),
            pl.BlockSpec((1, S, DK), lambda h, sb: (h // (H // KVH), 0, 0)),
            pl.BlockSpec((1, S, DK), lambda h, sb: (h // (H // KVH), 0, 0)),
        ],
        out_specs=pl.BlockSpec((1, QB, DK), lambda h, sb: (h, sb, 0)),
        out_shape=jax.ShapeDtypeStruct((H, S, DK), jnp.float32),
        compiler_params=pltpu.CompilerParams(
            dimension_semantics=("arbitrary", "arbitrary")),
    )(q, k, v)


def _k3_body(attn_ref, x_ref, wc_ref, g_ref, wr_ref, h_ref, hn_ref, pr_ref, eo_ref, wo_ref, hp_ref):
    attn = jnp.concatenate([attn_ref[hh] for hh in range(H)], axis=-1)
    h = x_ref[...] + jnp.dot(attn.astype(jnp.bfloat16), wc_ref[...],
                             preferred_element_type=jnp.float32)
    h_ref[...] = h
    hn = _rms(h, g_ref[...])
    hn_ref[...] = hn
    rl = jnp.dot(hn, wr_ref[...], preferred_element_type=jnp.float32)
    lane = jax.lax.broadcasted_iota(jnp.int32, rl.shape, 1)
    valid = lane < E
    rl = jnp.where(valid, rl, -1e30)
    m = jnp.max(rl, axis=-1, keepdims=True)
    p = jnp.exp(rl - m)
    p = p / jnp.sum(p, axis=-1, keepdims=True)
    pr_ref[...] = p
    m1 = jnp.max(p, axis=-1, keepdims=True)
    i1 = jnp.min(jnp.where(p == m1, lane, E), axis=-1, keepdims=True)
    p2 = jnp.where(valid & (lane != i1), p, -1.0)
    m2 = jnp.max(p2, axis=-1, keepdims=True)
    i2 = jnp.min(jnp.where(p2 == m2, lane, E), axis=-1, keepdims=True)
    tot = m1 + m2
    eo_ref[...] = jnp.where(lane == 0, i1, 0) + jnp.where(lane == 1, i2, 0)
    wo_ref[...] = jnp.where(lane == 0, m1 / tot, 0.0) + jnp.where(lane == 1, m2 / tot, 0.0)
    hnb = hn.astype(jnp.bfloat16)
    lo = lax.bitcast_convert_type(hnb[:, : D // 2], jnp.uint16).astype(jnp.uint32)
    hi = lax.bitcast_convert_type(hnb[:, D // 2 :], jnp.uint16).astype(jnp.uint32)
    hp_ref[...] = lax.bitcast_convert_type(lo | (hi << 16), jnp.int32)


def _k3(attn, x2, Wc, g_ff, Wr_pad):
    return pl.pallas_call(
        _k3_body,
        grid=(S // BS,),
        in_specs=[
            pl.BlockSpec((H, BS, DK), lambda i: (0, i, 0)),
            pl.BlockSpec((BS, D), lambda i: (i, 0)),
            pl.BlockSpec((D, D), lambda i: (0, 0)),
            pl.BlockSpec((D,), lambda i: (0,)),
            pl.BlockSpec((D, LANES), lambda i: (0, 0)),
        ],
        out_specs=[
            pl.BlockSpec((BS, D), lambda i: (i, 0)),
            pl.BlockSpec((BS, D), lambda i: (i, 0)),
            pl.BlockSpec((BS, LANES), lambda i: (i, 0)),
            pl.BlockSpec((BS, LANES), lambda i: (i, 0)),
            pl.BlockSpec((BS, LANES), lambda i: (i, 0)),
            pl.BlockSpec((BS, D // 2), lambda i: (i, 0)),
        ],
        out_shape=[
            jax.ShapeDtypeStruct((S, D), jnp.float32),
            jax.ShapeDtypeStruct((S, D), jnp.float32),
            jax.ShapeDtypeStruct((S, LANES), jnp.float32),
            jax.ShapeDtypeStruct((S, LANES), jnp.int32),
            jax.ShapeDtypeStruct((S, LANES), jnp.float32),
            jax.ShapeDtypeStruct((S, D // 2), jnp.int32),
        ],
        compiler_params=pltpu.CompilerParams(
            dimension_semantics=("arbitrary",)),
    )(attn, x2, Wc, g_ff, Wr_pad)


def _k5_body(hn_ref, h_ref, w1_ref, w2_ref, cp_ref, o_ref):
    hn = hn_ref[...].astype(jnp.bfloat16)
    a1 = jnp.dot(hn, w1_ref[...], preferred_element_type=jnp.float32)
    a2 = jnp.dot(hn, w2_ref[...], preferred_element_type=jnp.float32)
    act = (jax.nn.silu(a1) * a2).astype(jnp.bfloat16)
    o_ref[...] = h_ref[...] + jnp.dot(act, cp_ref[...],
                                      preferred_element_type=jnp.float32)


def _k5(hn, h, w1, w2, cp):
    return pl.pallas_call(
        _k5_body,
        grid=(S // BS,),
        in_specs=[
            pl.BlockSpec((BS, D), lambda i: (i, 0)),
            pl.BlockSpec((BS, D), lambda i: (i, 0)),
            pl.BlockSpec((D, HID), lambda i: (0, 0)),
            pl.BlockSpec((D, HID), lambda i: (0, 0)),
            pl.BlockSpec((HID, D), lambda i: (0, 0)),
        ],
        out_specs=pl.BlockSpec((BS, D), lambda i: (i, 0)),
        out_shape=jax.ShapeDtypeStruct((S, D), jnp.float32),
        compiler_params=pltpu.CompilerParams(
            dimension_semantics=("arbitrary",)),
    )(hn, h, w1, w2, cp)


def _sca_body(ef_hbm, wf_hbm, sids_hbm, sws_hbm, slots_hbm, bexp_hbm,
              ef_v, wf_v, sids_v, sws_v, slots_v, bexp_v):
    wid = lax.axis_index("s") * 2 + lax.axis_index("c")

    @pl.when(wid == 0)
    def _():
        pltpu.sync_copy(ef_hbm, ef_v)
        pltpu.sync_copy(wf_hbm, wf_v)

        def zinit(i, c):
            sids_v[pl.ds(i * 16, 16)] = jnp.zeros((16,), jnp.int32)
            sws_v[pl.ds(i * 16, 16)] = jnp.zeros((16,), jnp.float32)
            return c
        lax.fori_loop(0, NPAD // 16, zinit, 0)

        def p1(c, cnts):
            ids = ef_v[pl.ds(c * 16, 16)]
            return tuple(cnts[e] + jnp.sum((ids == e).astype(jnp.int32))
                         for e in range(E))
        cnts = lax.fori_loop(0, NP // 16, p1, (jnp.int32(0),) * E)

        pbs = []
        acc = jnp.int32(0)
        for e in range(E):
            pbs.append(acc)
            acc = acc + ((cnts[e] + (BM - 1)) // BM) * BM

        for cc in range(2):
            bidx = (lax.iota(jnp.int32, 16) + 16 * cc) * BM
            a = jnp.full((16,), -1, jnp.int32)
            for e in range(E):
                a = a + (bidx >= pbs[e]).astype(jnp.int32)
            bexp_v[pl.ds(cc * 16, 16)] = a

        def p2(c, cnts2):
            ids = ef_v[pl.ds(c * 16, 16)]
            w = wf_v[pl.ds(c * 16, 16)]
            p = lax.iota(jnp.int32, 16) + c * 16
            tok = jnp.bitwise_and(p, T - 1)
            slot = jnp.zeros((16,), jnp.int32)
            new = []
            for e in range(E):
                m = ids == e
                mi = m.astype(jnp.int32)
                cs = plsc.cumsum(mi)
                sv = pbs[e] + cnts2[e] + cs - 1
                slot = jnp.where(m, sv, slot)
                new.append(cnts2[e] + jnp.sum(mi))
            plsc.store_scatter(sids_v, [slot], tok)
            plsc.store_scatter(sws_v, [slot], w)
            slots_v[pl.ds(c * 16, 16)] = slot
            return tuple(new)
        lax.fori_loop(0, NP // 16, p2, (jnp.int32(0),) * E)

        pltpu.sync_copy(sids_v, sids_hbm)
        pltpu.sync_copy(sws_v, sws_hbm)
        pltpu.sync_copy(slots_v, slots_hbm)
        pltpu.sync_copy(bexp_v, bexp_hbm)


def _sca(eflat, wflat):
    mesh = plsc.VectorSubcoreMesh(core_axis_name="c", subcore_axis_name="s")
    f = pl.kernel(
        _sca_body,
        out_type=[
            jax.ShapeDtypeStruct((NPAD,), jnp.int32),
            jax.ShapeDtypeStruct((NPAD,), jnp.float32),
            jax.ShapeDtypeStruct((NP,), jnp.int32),
            jax.ShapeDtypeStruct((32,), jnp.int32),
        ],
        mesh=mesh,
        scratch_types=[
            pltpu.VMEM((NP,), jnp.int32),
            pltpu.VMEM((NP,), jnp.float32),
            pltpu.VMEM((NPAD,), jnp.int32),
            pltpu.VMEM((NPAD,), jnp.float32),
            pltpu.VMEM((NP,), jnp.int32),
            pltpu.VMEM((32,), jnp.int32),
        ],
        compiler_params=pltpu.CompilerParams(needs_layout_passes=False),
    )
    return f(eflat, wflat)


_RPW = NPAD // NW
_D2 = D // 2


_GCH = ((0, 24), (24, 24), (48, 24), (72, 24),
        (96, 24), (120, 24), (144, 24), (168, 16))


def _scb_body(hp_hbm, sids_hbm, xs_hbm, idx_v, buf_v, sem):
    wid = lax.axis_index("s") * 2 + lax.axis_index("c")
    base = wid * _RPW
    pltpu.sync_copy(sids_hbm.at[pl.ds(base, _RPW)], idx_v)
    hs = [pltpu.async_copy(hp_hbm.at[idx_v.at[pl.ds(o, n)]],
                           buf_v.at[pl.ds(o, n)], sem)
          for o, n in _GCH]
    for hnd in hs:
        hnd.wait()
    pltpu.sync_copy(buf_v, xs_hbm.at[pl.ds(base, _RPW)])


def _scb(hp, sids):
    mesh = plsc.VectorSubcoreMesh(core_axis_name="c", subcore_axis_name="s")
    f = pl.kernel(
        _scb_body,
        out_type=jax.ShapeDtypeStruct((NPAD, _D2), jnp.int32),
        mesh=mesh,
        scratch_types=[
            pltpu.VMEM((_RPW,), jnp.int32),
            pltpu.VMEM((_RPW, _D2), jnp.int32),
            pltpu.SemaphoreType.DMA,
        ],
        compiler_params=pltpu.CompilerParams(needs_layout_passes=False),
    )
    return f(hp, sids)


def _k6_body(bexp_ref, xs_ref, sw_ref, w1_ref, w2_ref, cp_ref, y_ref):
    u = lax.bitcast_convert_type(xs_ref[...], jnp.uint32)
    lo = lax.bitcast_convert_type((u & 0xFFFF).astype(jnp.uint16), jnp.bfloat16)
    hi = lax.bitcast_convert_type((u >> 16).astype(jnp.uint16), jnp.bfloat16)
    x = jnp.concatenate([lo, hi], axis=1)
    a1 = jnp.dot(x, w1_ref[0], preferred_element_type=jnp.float32)
    a2 = jnp.dot(x, w2_ref[0], preferred_element_type=jnp.float32)
    act = (jax.nn.silu(a1) * a2).astype(jnp.bfloat16)
    y = jnp.dot(act, cp_ref[0], preferred_element_type=jnp.float32)
    y_ref[...] = y * sw_ref[...]


def _k6(xs, sws_col, bexp, ew1, ew2, ecp):
    grid_spec = pltpu.PrefetchScalarGridSpec(
        num_scalar_prefetch=1,
        grid=(NBLK,),
        in_specs=[
            pl.BlockSpec((BM, _D2), lambda b, be: (b, 0)),
            pl.BlockSpec((BM, 1), lambda b, be: (b, 0)),
            pl.BlockSpec((1, D, HID), lambda b, be: (be[b], 0, 0)),
            pl.BlockSpec((1, D, HID), lambda b, be: (be[b], 0, 0)),
            pl.BlockSpec((1, HID, D), lambda b, be: (be[b], 0, 0)),
        ],
        out_specs=pl.BlockSpec((BM, D), lambda b, be: (b, 0)),
    )
    return pl.pallas_call(
        _k6_body,
        grid_spec=grid_spec,
        out_shape=jax.ShapeDtypeStruct((NPAD, D), jnp.float32),
        compiler_params=pltpu.CompilerParams(
            dimension_semantics=("arbitrary",)),
    )(bexp, xs, sws_col, ew1, ew2, ecp)


_TPW = T // NW


def _scc_body(y_hbm, base_hbm, slots_hbm, out_hbm,
              i0_v, i1_v, y0_v, y1_v, b_v, o_v, sem0, sem1, semb):
    wid = lax.axis_index("s") * 2 + lax.axis_index("c")
    t0 = wid * _TPW
    pltpu.sync_copy(slots_hbm.at[pl.ds(t0, _TPW)], i0_v)
    pltpu.sync_copy(slots_hbm.at[pl.ds(T + t0, _TPW)], i1_v)
    for s in range(_TPW // 16):
        i0 = i0_v[pl.ds(s * 16, 16)]
        i1 = i1_v[pl.ds(s * 16, 16)]
        cp0 = pltpu.async_copy(y_hbm.at[i0], y0_v, sem0)
        cp1 = pltpu.async_copy(y_hbm.at[i1], y1_v, sem1)
        cpb = pltpu.async_copy(base_hbm.at[pl.ds(t0 + s * 16, 16)], b_v, semb)
        cp0.wait()
        cp1.wait()
        cpb.wait()
        for i in range(16):
            def add_j(j, c):
                sl = pl.ds(pl.multiple_of(j * 16, 16), 16)
                o_v[i, sl] = b_v[i, sl] + y0_v[i, sl] + y1_v[i, sl]
                return c
            lax.fori_loop(0, D // 16, add_j, 0)
        pltpu.sync_copy(o_v, out_hbm.at[pl.ds(t0 + s * 16, 16)])


def _scc(y, base, slots):
    mesh = plsc.VectorSubcoreMesh(core_axis_name="c", subcore_axis_name="s")
    f = pl.kernel(
        _scc_body,
        out_type=jax.ShapeDtypeStruct((T, D), jnp.float32),
        mesh=mesh,
        scratch_types=[
            pltpu.VMEM((_TPW,), jnp.int32),
            pltpu.VMEM((_TPW,), jnp.int32),
            pltpu.VMEM((16, D), jnp.float32),
            pltpu.VMEM((16, D), jnp.float32),
            pltpu.VMEM((16, D), jnp.float32),
            pltpu.VMEM((16, D), jnp.float32),
            pltpu.SemaphoreType.DMA,
            pltpu.SemaphoreType.DMA,
            pltpu.SemaphoreType.DMA,
        ],
        compiler_params=pltpu.CompilerParams(needs_layout_passes=False),
    )
    return f(y, base, slots)


def kernel(x, g_attn, Wq, Wkv, Wc, g_ff, Wr, sw1, sw2, scp, ew1, ew2, ecp,
           start_posn=0, use_kv_cache=False):
    x2 = x.reshape(S, D)
    pairs = jnp.arange(DK // 2, dtype=jnp.float32)
    freqs = 1.0 / THETA ** (2.0 * pairs / DK)
    pos = jnp.arange(S, dtype=jnp.float32) + jnp.asarray(start_posn, jnp.float32)
    ang = pos[:, None] * freqs[None, :]
    cos2 = jnp.repeat(jnp.cos(ang), 2, axis=1)
    sgn = jnp.tile(jnp.array([-1.0, 1.0], jnp.float32), DK // 2)
    sin2 = jnp.repeat(jnp.sin(ang), 2, axis=1) * sgn[None, :]
    cosD = jnp.tile(cos2, (1, H))
    sinD = jnp.tile(sin2, (1, H))

    Wr_pad = jnp.zeros((D, LANES), jnp.float32).at[:, :E].set(Wr)

    q, k, v = _k1(x2, g_attn, Wq.astype(jnp.bfloat16),
                  Wkv.astype(jnp.bfloat16), cosD, sinD)
    attn = _k2(q, k, v)
    h, hn, probs_pad, eo, wo, hp = _k3(attn, x2, Wc.astype(jnp.bfloat16), g_ff,
                                       Wr_pad)
    eflat = jnp.concatenate([eo[:, 0], eo[:, 1]])
    wflat = jnp.concatenate([wo[:, 0], wo[:, 1]])
    sids, sws, slots, bexp = _sca(eflat, wflat)
    xs = _scb(hp, sids)
    base = _k5(hn, h, sw1[0].astype(jnp.bfloat16), sw2[0].astype(jnp.bfloat16),
               scp[0].astype(jnp.bfloat16))
    y = _k6(xs, sws.reshape(NPAD, 1), bexp, ew1.astype(jnp.bfloat16),
            ew2.astype(jnp.bfloat16), ecp.astype(jnp.bfloat16))
    out = _scc(y, base, slots)
    return out.reshape(B, S, D), probs_pad[:, :E].reshape(B, S, E)


# speedup vs baseline: 1.1577x; 1.1577x over previous
import functools
import math

import jax
import jax.numpy as jnp
from jax import lax
from jax.experimental import pallas as pl
from jax.experimental.pallas import tpu as pltpu
from jax.experimental.pallas import tpu_sc as plsc

B = 1
S = 2048
D = 1024
H = 16
KVH = 8
DK = D // H
HID = 1024
E = 8
TOPK = 2
NSH = 1
THETA = 10000.0
EPS = 1e-6

BS = 256
QB = 512
LANES = 128

T = S
NP = T * TOPK
BM = 256
NPAD = 5888
NBLK = NPAD // BM
NW = 32


def _rms(x, g):
    return g * (x / jnp.sqrt(jnp.mean(x * x, axis=-1, keepdims=True) + EPS))


def _k1_body(x_ref, g_ref, wq_ref, wkv_ref, cos_ref, sin_ref, q_ref, k_ref, v_ref):
    xn = _rms(x_ref[...], g_ref[...]).astype(jnp.bfloat16)
    q = jnp.dot(xn, wq_ref[...], preferred_element_type=jnp.float32)
    kv = jnp.dot(xn, wkv_ref[...], preferred_element_type=jnp.float32)
    k = kv[:, : KVH * DK]
    v = kv[:, KVH * DK :]
    cos = cos_ref[...]
    sin = sin_ref[...]

    def rope(t, c, s):
        even = jax.lax.broadcasted_iota(jnp.int32, t.shape, 1) % 2 == 0
        n = t.shape[1]
        swap = jnp.where(even, pltpu.roll(t, n - 1, 1), pltpu.roll(t, 1, 1))
        return t * c + swap * s

    qr = rope(q, cos, sin) * (1.0 / math.sqrt(DK))
    kr = rope(k, cos[:, : KVH * DK], sin[:, : KVH * DK])
    for hh in range(H):
        q_ref[hh] = qr[:, hh * DK : (hh + 1) * DK]
    for hh in range(KVH):
        k_ref[hh] = kr[:, hh * DK : (hh + 1) * DK]
        v_ref[hh] = v[:, hh * DK : (hh + 1) * DK]


def _k1(x2, g_attn, Wq, Wkv, cosD, sinD):
    return pl.pallas_call(
        _k1_body,
        grid=(S // BS,),
        in_specs=[
            pl.BlockSpec((BS, D), lambda i: (i, 0)),
            pl.BlockSpec((D,), lambda i: (0,)),
            pl.BlockSpec((D, D), lambda i: (0, 0)),
            pl.BlockSpec((D, D), lambda i: (0, 0)),
            pl.BlockSpec((BS, D), lambda i: (i, 0)),
            pl.BlockSpec((BS, D), lambda i: (i, 0)),
        ],
        out_specs=[
            pl.BlockSpec((H, BS, DK), lambda i: (0, i, 0)),
            pl.BlockSpec((KVH, BS, DK), lambda i: (0, i, 0)),
            pl.BlockSpec((KVH, BS, DK), lambda i: (0, i, 0)),
        ],
        out_shape=[
            jax.ShapeDtypeStruct((H, S, DK), jnp.float32),
            jax.ShapeDtypeStruct((KVH, S, DK), jnp.float32),
            jax.ShapeDtypeStruct((KVH, S, DK), jnp.float32),
        ],
        compiler_params=pltpu.CompilerParams(
            dimension_semantics=("arbitrary",)),
    )(x2, g_attn, Wq, Wkv, cosD, sinD)


def _k2_body(q_ref, k_ref, v_ref, o_ref):
    sb = pl.program_id(1)
    q = q_ref[0]
    k = k_ref[0]
    v = v_ref[0]
    logits = jax.lax.dot_general(q, k, (((1,), (1,)), ((), ())),
                                 preferred_element_type=jnp.float32)
    row = jax.lax.broadcasted_iota(jnp.int32, logits.shape, 0) + sb * QB
    col = jax.lax.broadcasted_iota(jnp.int32, logits.shape, 1)
    logits = jnp.where(col <= row, logits, -1e30)
    m = jnp.max(logits, axis=-1, keepdims=True)
    p = jnp.exp(logits - m)
    p = p / jnp.sum(p, axis=-1, keepdims=True)
    o_ref[0] = jnp.dot(p, v, preferred_element_type=jnp.float32)


def _k2(q, k, v):
    return pl.pallas_call(
        _k2_body,
        grid=(H, S // QB),
        in_specs=[
            pl.BlockSpec((1, QB, DK), lambda h, sb: (h, sb, 0)),
            pl.BlockSpec((1, S, DK), lambda h, sb: (h // (H // KVH), 0, 0)),
            pl.BlockSpec((1, S, DK), lambda h, sb: (h // (H // KVH), 0, 0)),
        ],
        out_specs=pl.BlockSpec((1, QB, DK), lambda h, sb: (h, sb, 0)),
        out_shape=jax.ShapeDtypeStruct((H, S, DK), jnp.float32),
        compiler_params=pltpu.CompilerParams(
            dimension_semantics=("arbitrary", "arbitrary")),
    )(q, k, v)


def _k3_body(attn_ref, x_ref, wc_ref, g_ref, wr_ref, h_ref, hn_ref, pr_ref, eo_ref, wo_ref, hp_ref, cw_ref):
    attn = jnp.concatenate([attn_ref[hh] for hh in range(H)], axis=-1)
    h = x_ref[...] + jnp.dot(attn.astype(jnp.bfloat16), wc_ref[...],
                             preferred_element_type=jnp.float32)
    h_ref[...] = h
    hn = _rms(h, g_ref[...])
    hn_ref[...] = hn
    rl = jnp.dot(hn, wr_ref[...], preferred_element_type=jnp.float32)
    lane = jax.lax.broadcasted_iota(jnp.int32, rl.shape, 1)
    valid = lane < E
    rl = jnp.where(valid, rl, -1e30)
    m = jnp.max(rl, axis=-1, keepdims=True)
    p = jnp.exp(rl - m)
    p = p / jnp.sum(p, axis=-1, keepdims=True)
    pr_ref[...] = p
    m1 = jnp.max(p, axis=-1, keepdims=True)
    i1 = jnp.min(jnp.where(p == m1, lane, E), axis=-1, keepdims=True)
    p2 = jnp.where(valid & (lane != i1), p, -1.0)
    m2 = jnp.max(p2, axis=-1, keepdims=True)
    i2 = jnp.min(jnp.where(p2 == m2, lane, E), axis=-1, keepdims=True)
    tot = m1 + m2
    eo_ref[...] = jnp.where(lane == 0, i1, 0) + jnp.where(lane == 1, i2, 0)
    wo_ref[...] = jnp.where(lane == 0, m1 / tot, 0.0) + jnp.where(lane == 1, m2 / tot, 0.0)
    cw_ref[...] = (jnp.where(lane == i1, m1 / tot, 0.0)
                   + jnp.where(lane == i2, m2 / tot, 0.0))
    hnb = hn.astype(jnp.bfloat16)
    lo = lax.bitcast_convert_type(hnb[:, : D // 2], jnp.uint16).astype(jnp.uint32)
    hi = lax.bitcast_convert_type(hnb[:, D // 2 :], jnp.uint16).astype(jnp.uint32)
    hp_ref[...] = lax.bitcast_convert_type(lo | (hi << 16), jnp.int32)


def _k3(attn, x2, Wc, g_ff, Wr_pad):
    return pl.pallas_call(
        _k3_body,
        grid=(S // BS,),
        in_specs=[
            pl.BlockSpec((H, BS, DK), lambda i: (0, i, 0)),
            pl.BlockSpec((BS, D), lambda i: (i, 0)),
            pl.BlockSpec((D, D), lambda i: (0, 0)),
            pl.BlockSpec((D,), lambda i: (0,)),
            pl.BlockSpec((D, LANES), lambda i: (0, 0)),
        ],
        out_specs=[
            pl.BlockSpec((BS, D), lambda i: (i, 0)),
            pl.BlockSpec((BS, D), lambda i: (i, 0)),
            pl.BlockSpec((BS, LANES), lambda i: (i, 0)),
            pl.BlockSpec((BS, LANES), lambda i: (i, 0)),
            pl.BlockSpec((BS, LANES), lambda i: (i, 0)),
            pl.BlockSpec((BS, D // 2), lambda i: (i, 0)),
            pl.BlockSpec((BS, LANES), lambda i: (i, 0)),
        ],
        out_shape=[
            jax.ShapeDtypeStruct((S, D), jnp.float32),
            jax.ShapeDtypeStruct((S, D), jnp.float32),
            jax.ShapeDtypeStruct((S, LANES), jnp.float32),
            jax.ShapeDtypeStruct((S, LANES), jnp.int32),
            jax.ShapeDtypeStruct((S, LANES), jnp.float32),
            jax.ShapeDtypeStruct((S, D // 2), jnp.int32),
            jax.ShapeDtypeStruct((S, LANES), jnp.float32),
        ],
        compiler_params=pltpu.CompilerParams(
            dimension_semantics=("arbitrary",)),
    )(attn, x2, Wc, g_ff, Wr_pad)


def _k5_body(hn_ref, h_ref, w1_ref, w2_ref, cp_ref, o_ref):
    hn = hn_ref[...].astype(jnp.bfloat16)
    a1 = jnp.dot(hn, w1_ref[...], preferred_element_type=jnp.float32)
    a2 = jnp.dot(hn, w2_ref[...], preferred_element_type=jnp.float32)
    act = (jax.nn.silu(a1) * a2).astype(jnp.bfloat16)
    o_ref[...] = h_ref[...] + jnp.dot(act, cp_ref[...],
                                      preferred_element_type=jnp.float32)


def _k5(hn, h, w1, w2, cp):
    return pl.pallas_call(
        _k5_body,
        grid=(S // BS,),
        in_specs=[
            pl.BlockSpec((BS, D), lambda i: (i, 0)),
            pl.BlockSpec((BS, D), lambda i: (i, 0)),
            pl.BlockSpec((D, HID), lambda i: (0, 0)),
            pl.BlockSpec((D, HID), lambda i: (0, 0)),
            pl.BlockSpec((HID, D), lambda i: (0, 0)),
        ],
        out_specs=pl.BlockSpec((BS, D), lambda i: (i, 0)),
        out_shape=jax.ShapeDtypeStruct((S, D), jnp.float32),
        compiler_params=pltpu.CompilerParams(
            dimension_semantics=("arbitrary",)),
    )(hn, h, w1, w2, cp)


def _sca_body(ef_hbm, wf_hbm, sids_hbm, sws_hbm, slots_hbm, bexp_hbm,
              ef_v, wf_v, sids_v, sws_v, slots_v, bexp_v):
    wid = lax.axis_index("s") * 2 + lax.axis_index("c")

    @pl.when(wid == 0)
    def _():
        pltpu.sync_copy(ef_hbm, ef_v)
        pltpu.sync_copy(wf_hbm, wf_v)

        def zinit(i, c):
            sids_v[pl.ds(i * 16, 16)] = jnp.zeros((16,), jnp.int32)
            sws_v[pl.ds(i * 16, 16)] = jnp.zeros((16,), jnp.float32)
            return c
        lax.fori_loop(0, NPAD // 16, zinit, 0)

        def p1(c, cnts):
            ids = ef_v[pl.ds(c * 16, 16)]
            return tuple(cnts[e] + jnp.sum((ids == e).astype(jnp.int32))
                         for e in range(E))
        cnts = lax.fori_loop(0, NP // 16, p1, (jnp.int32(0),) * E)

        pbs = []
        acc = jnp.int32(0)
        for e in range(E):
            pbs.append(acc)
            acc = acc + ((cnts[e] + (BM - 1)) // BM) * BM

        for cc in range(2):
            bidx = (lax.iota(jnp.int32, 16) + 16 * cc) * BM
            a = jnp.full((16,), -1, jnp.int32)
            for e in range(E):
                a = a + (bidx >= pbs[e]).astype(jnp.int32)
            bexp_v[pl.ds(cc * 16, 16)] = a

        def p2(c, cnts2):
            ids = ef_v[pl.ds(c * 16, 16)]
            w = wf_v[pl.ds(c * 16, 16)]
            p = lax.iota(jnp.int32, 16) + c * 16
            tok = jnp.bitwise_and(p, T - 1)
            slot = jnp.zeros((16,), jnp.int32)
            new = []
            for e in range(E):
                m = ids == e
                mi = m.astype(jnp.int32)
                cs = plsc.cumsum(mi)
                sv = pbs[e] + cnts2[e] + cs - 1
                slot = jnp.where(m, sv, slot)
                new.append(cnts2[e] + jnp.sum(mi))
            plsc.store_scatter(sids_v, [slot], tok)
            plsc.store_scatter(sws_v, [slot], w)
            slots_v[pl.ds(c * 16, 16)] = slot
            return tuple(new)
        lax.fori_loop(0, NP // 16, p2, (jnp.int32(0),) * E)

        pltpu.sync_copy(sids_v, sids_hbm)
        pltpu.sync_copy(sws_v, sws_hbm)
        pltpu.sync_copy(slots_v, slots_hbm)
        pltpu.sync_copy(bexp_v, bexp_hbm)


def _sca(eflat, wflat):
    mesh = plsc.VectorSubcoreMesh(core_axis_name="c", subcore_axis_name="s")
    f = pl.kernel(
        _sca_body,
        out_type=[
            jax.ShapeDtypeStruct((NPAD,), jnp.int32),
            jax.ShapeDtypeStruct((NPAD,), jnp.float32),
            jax.ShapeDtypeStruct((NP,), jnp.int32),
            jax.ShapeDtypeStruct((32,), jnp.int32),
        ],
        mesh=mesh,
        scratch_types=[
            pltpu.VMEM((NP,), jnp.int32),
            pltpu.VMEM((NP,), jnp.float32),
            pltpu.VMEM((NPAD,), jnp.int32),
            pltpu.VMEM((NPAD,), jnp.float32),
            pltpu.VMEM((NP,), jnp.int32),
            pltpu.VMEM((32,), jnp.int32),
        ],
        compiler_params=pltpu.CompilerParams(needs_layout_passes=False),
    )
    return f(eflat, wflat)


_RPW = NPAD // NW
_D2 = D // 2


_GCH = ((0, 24), (24, 24), (48, 24), (72, 24),
        (96, 24), (120, 24), (144, 24), (168, 16))


def _scb_body(hp_hbm, sids_hbm, xs_hbm, idx_v, buf_v, sem):
    wid = lax.axis_index("s") * 2 + lax.axis_index("c")
    base = wid * _RPW
    pltpu.sync_copy(sids_hbm.at[pl.ds(base, _RPW)], idx_v)
    hs = [pltpu.async_copy(hp_hbm.at[idx_v.at[pl.ds(o, n)]],
                           buf_v.at[pl.ds(o, n)], sem)
          for o, n in _GCH]
    for hnd in hs:
        hnd.wait()
    pltpu.sync_copy(buf_v, xs_hbm.at[pl.ds(base, _RPW)])


def _scb(hp, sids):
    mesh = plsc.VectorSubcoreMesh(core_axis_name="c", subcore_axis_name="s")
    f = pl.kernel(
        _scb_body,
        out_type=jax.ShapeDtypeStruct((NPAD, _D2), jnp.int32),
        mesh=mesh,
        scratch_types=[
            pltpu.VMEM((_RPW,), jnp.int32),
            pltpu.VMEM((_RPW, _D2), jnp.int32),
            pltpu.SemaphoreType.DMA,
        ],
        compiler_params=pltpu.CompilerParams(needs_layout_passes=False),
    )
    return f(hp, sids)


def _k6_body(bexp_ref, xs_ref, sw_ref, w1_ref, w2_ref, cp_ref, y_ref):
    u = lax.bitcast_convert_type(xs_ref[...], jnp.uint32)
    lo = lax.bitcast_convert_type((u & 0xFFFF).astype(jnp.uint16), jnp.bfloat16)
    hi = lax.bitcast_convert_type((u >> 16).astype(jnp.uint16), jnp.bfloat16)
    x = jnp.concatenate([lo, hi], axis=1)
    a1 = jnp.dot(x, w1_ref[0], preferred_element_type=jnp.float32)
    a2 = jnp.dot(x, w2_ref[0], preferred_element_type=jnp.float32)
    act = (jax.nn.silu(a1) * a2).astype(jnp.bfloat16)
    y = jnp.dot(act, cp_ref[0], preferred_element_type=jnp.float32)
    y_ref[...] = y * sw_ref[...]


def _k6(xs, sws_col, bexp, ew1, ew2, ecp):
    grid_spec = pltpu.PrefetchScalarGridSpec(
        num_scalar_prefetch=1,
        grid=(NBLK,),
        in_specs=[
            pl.BlockSpec((BM, _D2), lambda b, be: (b, 0)),
            pl.BlockSpec((BM, 1), lambda b, be: (b, 0)),
            pl.BlockSpec((1, D, HID), lambda b, be: (be[b], 0, 0)),
            pl.BlockSpec((1, D, HID), lambda b, be: (be[b], 0, 0)),
            pl.BlockSpec((1, HID, D), lambda b, be: (be[b], 0, 0)),
        ],
        out_specs=pl.BlockSpec((BM, D), lambda b, be: (b, 0)),
    )
    return pl.pallas_call(
        _k6_body,
        grid_spec=grid_spec,
        out_shape=jax.ShapeDtypeStruct((NPAD, D), jnp.float32),
        compiler_params=pltpu.CompilerParams(
            dimension_semantics=("arbitrary",)),
    )(bexp, xs, sws_col, ew1, ew2, ecp)


def _k4d_body(hn_ref, base_ref, cw_ref, w1_ref, w2_ref, cp_ref, o_ref):
    e = pl.program_id(0)
    sb = pl.program_id(1)
    rows = pl.ds(sb * BS, BS)
    hn = hn_ref[rows, :].astype(jnp.bfloat16)
    cwb = cw_ref[rows, :]
    lane = jax.lax.broadcasted_iota(jnp.int32, cwb.shape, 1)
    cwcol = jnp.sum(jnp.where(lane == e, cwb, 0.0), axis=-1, keepdims=True)
    a1 = jnp.dot(hn, w1_ref[0], preferred_element_type=jnp.float32)
    a2 = jnp.dot(hn, w2_ref[0], preferred_element_type=jnp.float32)
    act = (jax.nn.silu(a1) * a2).astype(jnp.bfloat16)
    contrib = cwcol * jnp.dot(act, cp_ref[0], preferred_element_type=jnp.float32)

    @pl.when(e == 0)
    def _():
        o_ref[rows, :] = base_ref[rows, :] + contrib

    @pl.when(e > 0)
    def _():
        o_ref[rows, :] = o_ref[rows, :] + contrib


def _k4d(hn, base, cw_pad, ew1, ew2, ecp):
    full = lambda e, sb: (0, 0)
    return pl.pallas_call(
        _k4d_body,
        grid=(E, S // BS),
        in_specs=[
            pl.BlockSpec((S, D), full),
            pl.BlockSpec((S, D), full),
            pl.BlockSpec((S, LANES), full),
            pl.BlockSpec((1, D, HID), lambda e, sb: (e, 0, 0)),
            pl.BlockSpec((1, D, HID), lambda e, sb: (e, 0, 0)),
            pl.BlockSpec((1, HID, D), lambda e, sb: (e, 0, 0)),
        ],
        out_specs=pl.BlockSpec((S, D), full),
        out_shape=jax.ShapeDtypeStruct((S, D), jnp.float32),
        compiler_params=pltpu.CompilerParams(
            dimension_semantics=("arbitrary", "arbitrary")),
    )(hn, base, cw_pad, ew1, ew2, ecp)


_TPW = T // NW


def _scc_body(y_hbm, base_hbm, slots_hbm, out_hbm,
              i0_v, i1_v, y0_v, y1_v, b_v, o_v, sem0, sem1, semb):
    wid = lax.axis_index("s") * 2 + lax.axis_index("c")
    t0 = wid * _TPW
    pltpu.sync_copy(slots_hbm.at[pl.ds(t0, _TPW)], i0_v)
    pltpu.sync_copy(slots_hbm.at[pl.ds(T + t0, _TPW)], i1_v)
    for s in range(_TPW // 16):
        i0 = i0_v[pl.ds(s * 16, 16)]
        i1 = i1_v[pl.ds(s * 16, 16)]
        cp0 = pltpu.async_copy(y_hbm.at[i0], y0_v, sem0)
        cp1 = pltpu.async_copy(y_hbm.at[i1], y1_v, sem1)
        cpb = pltpu.async_copy(base_hbm.at[pl.ds(t0 + s * 16, 16)], b_v, semb)
        cp0.wait()
        cp1.wait()
        cpb.wait()
        for i in range(16):
            def add_j(j, c):
                sl = pl.ds(pl.multiple_of(j * 16, 16), 16)
                o_v[i, sl] = b_v[i, sl] + y0_v[i, sl] + y1_v[i, sl]
                return c
            lax.fori_loop(0, D // 16, add_j, 0)
        pltpu.sync_copy(o_v, out_hbm.at[pl.ds(t0 + s * 16, 16)])


def _scc(y, base, slots):
    mesh = plsc.VectorSubcoreMesh(core_axis_name="c", subcore_axis_name="s")
    f = pl.kernel(
        _scc_body,
        out_type=jax.ShapeDtypeStruct((T, D), jnp.float32),
        mesh=mesh,
        scratch_types=[
            pltpu.VMEM((_TPW,), jnp.int32),
            pltpu.VMEM((_TPW,), jnp.int32),
            pltpu.VMEM((16, D), jnp.float32),
            pltpu.VMEM((16, D), jnp.float32),
            pltpu.VMEM((16, D), jnp.float32),
            pltpu.VMEM((16, D), jnp.float32),
            pltpu.SemaphoreType.DMA,
            pltpu.SemaphoreType.DMA,
            pltpu.SemaphoreType.DMA,
        ],
        compiler_params=pltpu.CompilerParams(needs_layout_passes=False),
    )
    return f(y, base, slots)


def kernel(x, g_attn, Wq, Wkv, Wc, g_ff, Wr, sw1, sw2, scp, ew1, ew2, ecp,
           start_posn=0, use_kv_cache=False):
    x2 = x.reshape(S, D)
    pairs = jnp.arange(DK // 2, dtype=jnp.float32)
    freqs = 1.0 / THETA ** (2.0 * pairs / DK)
    pos = jnp.arange(S, dtype=jnp.float32) + jnp.asarray(start_posn, jnp.float32)
    ang = pos[:, None] * freqs[None, :]
    cos2 = jnp.repeat(jnp.cos(ang), 2, axis=1)
    sgn = jnp.tile(jnp.array([-1.0, 1.0], jnp.float32), DK // 2)
    sin2 = jnp.repeat(jnp.sin(ang), 2, axis=1) * sgn[None, :]
    cosD = jnp.tile(cos2, (1, H))
    sinD = jnp.tile(sin2, (1, H))

    Wr_pad = jnp.zeros((D, LANES), jnp.float32).at[:, :E].set(Wr)

    q, k, v = _k1(x2, g_attn, Wq.astype(jnp.bfloat16),
                  Wkv.astype(jnp.bfloat16), cosD, sinD)
    attn = _k2(q, k, v)
    h, hn, probs_pad, eo, wo, hp, cw_pad = _k3(attn, x2,
                                               Wc.astype(jnp.bfloat16), g_ff,
                                               Wr_pad)
    base = _k5(hn, h, sw1[0].astype(jnp.bfloat16), sw2[0].astype(jnp.bfloat16),
               scp[0].astype(jnp.bfloat16))
    out = _k4d(hn, base, cw_pad, ew1.astype(jnp.bfloat16),
               ew2.astype(jnp.bfloat16), ecp.astype(jnp.bfloat16))
    return out.reshape(B, S, D), probs_pad[:, :E].reshape(B, S, E)
